# Initial kernel scaffold; baseline (speedup 1.0000x reference)
import jax
import jax.numpy as jnp
from jax.experimental import pallas as pl


def kernel(h, edge_index_r0, edge_index_r1, edge_index_r2, relation_bias, W, node_query, rel_vector):
    raise NotImplementedError("write your pallas kernel here")


# speedup vs baseline: 12.2863x; 12.2863x over previous
import functools

import jax
import jax.numpy as jnp
from jax import lax
from jax.experimental import pallas as pl
from jax.experimental.pallas import tpu as pltpu
from jax.experimental.pallas import tpu_sc as plsc

_N = 10000
_E = 320000
_D = 128
_NREL = 3
_NHEADS = 4

_C = 128
_NCHUNK = _E // _C
_NW = 32
_FULL_J = _NCHUNK // _NW
_EXTRA = _NCHUNK - _FULL_J * _NW
_RPT = 640
_NPAD = 16 * _RPT

_BN = 1000


def _dense_body(h_ref, w_ref, q_ref, out_ref):
    hb = h_ref[...]
    for r in range(_NREL):
        acc = None
        qr = q_ref[r]
        for hd in range(_NHEADS):
            wm = w_ref[r, hd]
            hp = lax.dot_general(hb, wm, (((1,), (1,)), ((), ())),
                                 preferred_element_type=jnp.float32)
            qv = qr[hd:hd + 1, :]
            g = jnp.sum(hp * qv, axis=1, keepdims=True)
            gate = jax.nn.sigmoid(g)
            term = hp * gate
            acc = term if acc is None else acc + term
        out_ref[r] = acc * (1.0 / _NHEADS)


def _dense_call(h, W, q_pad):
    grid = _N // _BN
    return pl.pallas_call(
        _dense_body,
        grid=(grid,),
        in_specs=[
            pl.BlockSpec((_BN, _D), lambda i: (i, 0)),
            pl.BlockSpec((_NREL, _NHEADS, _D, _D), lambda i: (0, 0, 0, 0)),
            pl.BlockSpec((_NREL, 8, _D), lambda i: (0, 0, 0)),
        ],
        out_specs=pl.BlockSpec((_NREL, _BN, _D), lambda i: (0, i, 0)),
        out_shape=jax.ShapeDtypeStruct((_NREL, _N, _D), jnp.float32),
    )(h, W, q_pad)


def _sc_body(xbar_hbm, srcs_hbm, dsts_hbm, zeros_hbm, out_hbm,
             acc, rows_v, src_v, dst_v, sem):
    c = lax.axis_index("c")
    s = lax.axis_index("s")
    wid = s * 2 + c
    base = s * _RPT

    for r in range(_NREL):
        pltpu.sync_copy(zeros_hbm, acc.at[pl.ds(base, _RPT)])
        plsc.subcore_barrier()

        def chunk(j, carry, extra_cid=None):
            cid = r * _NCHUNK + (j * _NW + wid if extra_cid is None else extra_cid)
            pltpu.sync_copy(srcs_hbm.at[cid], src_v)
            pltpu.sync_copy(dsts_hbm.at[cid], dst_v)
            pltpu.async_copy(xbar_hbm.at[src_v], rows_v, sem).wait()
            pltpu.sync_copy(rows_v, acc.at[dst_v], add=True)
            return carry

        lax.fori_loop(0, _FULL_J, chunk, 0)

        @pl.when(wid < _EXTRA)
        def _():
            chunk(0, 0, extra_cid=_FULL_J * _NW + wid)

        plsc.subcore_barrier()

        out_off = (c * _NREL + r) * _NPAD + base
        pltpu.sync_copy(acc.at[pl.ds(base, _RPT)],
                        out_hbm.at[pl.ds(out_off, _RPT)])
        plsc.subcore_barrier()


def _sc_call(xbar_flat, srcs, dsts, zeros_hbm):
    mesh = plsc.VectorSubcoreMesh(core_axis_name="c", subcore_axis_name="s")
    return pl.kernel(
        _sc_body,
        out_type=jax.ShapeDtypeStruct((2 * _NREL * _NPAD, _D), jnp.float32),
        mesh=mesh,
        scratch_types=[
            pltpu.VMEM_SHARED((_NPAD, _D), jnp.float32),
            pltpu.VMEM((_C, _D), jnp.float32),
            pltpu.VMEM((_C,), jnp.int32),
            pltpu.VMEM((_C,), jnp.int32),
            pltpu.SemaphoreType.DMA,
        ],
    )(xbar_flat, srcs, dsts, zeros_hbm)


def _fuse_body(p_ref, rv_ref, bias_ref, h_ref, attn_ref):
    m0 = p_ref[0] + p_ref[3]
    m1 = p_ref[1] + p_ref[4]
    m2 = p_ref[2] + p_ref[5]
    rv = rv_ref[...]
    l0 = jnp.sum(m0 * rv[0:1, :], axis=1, keepdims=True) + bias_ref[0]
    l1 = jnp.sum(m1 * rv[1:2, :], axis=1, keepdims=True) + bias_ref[1]
    l2 = jnp.sum(m2 * rv[2:3, :], axis=1, keepdims=True) + bias_ref[2]
    mx = jnp.maximum(jnp.maximum(l0, l1), l2)
    e0 = jnp.exp(l0 - mx)
    e1 = jnp.exp(l1 - mx)
    e2 = jnp.exp(l2 - mx)
    inv = 1.0 / (e0 + e1 + e2)
    a0 = e0 * inv
    a1 = e1 * inv
    a2 = e2 * inv
    h_ref[...] = jnp.maximum(a0 * m0 + a1 * m1 + a2 * m2, 0.0)
    col = lax.broadcasted_iota(jnp.int32, (1, _D), 1)
    attn_ref[0] = jnp.where(
        col == 0, jnp.sum(a0),
        jnp.where(col == 1, jnp.sum(a1),
                  jnp.where(col == 2, jnp.sum(a2), 0.0)))


def _fuse_call(partials, rv_pad, relation_bias):
    grid = _N // _BN
    return pl.pallas_call(
        _fuse_body,
        grid=(grid,),
        in_specs=[
            pl.BlockSpec((2 * _NREL, _BN, _D), lambda i: (0, i, 0)),
            pl.BlockSpec((8, _D), lambda i: (0, 0)),
            pl.BlockSpec(memory_space=pltpu.SMEM),
        ],
        out_specs=[
            pl.BlockSpec((_BN, _D), lambda i: (i, 0)),
            pl.BlockSpec((1, 1, _D), lambda i: (i, 0, 0)),
        ],
        out_shape=[
            jax.ShapeDtypeStruct((_N, _D), jnp.float32),
            jax.ShapeDtypeStruct((grid, 1, _D), jnp.float32),
        ],
    )(partials, rv_pad, relation_bias)


def kernel(h, edge_index_r0, edge_index_r1, edge_index_r2,
           relation_bias, W, node_query, rel_vector):
    q_pad = jnp.zeros((_NREL, 8, _D), jnp.float32).at[:, :_NHEADS].set(node_query)
    rv_pad = jnp.zeros((8, _D), jnp.float32).at[:_NREL].set(rel_vector)

    src_all = jnp.stack([edge_index_r0[1], edge_index_r1[1], edge_index_r2[1]])
    src_all = src_all + (jnp.arange(_NREL, dtype=jnp.int32) * _N)[:, None]
    srcs = src_all.reshape(_NREL * _NCHUNK, _C)
    dsts = jnp.stack([edge_index_r0[0], edge_index_r1[0], edge_index_r2[0]])
    dsts = dsts.reshape(_NREL * _NCHUNK, _C)
    zeros_hbm = jnp.zeros((_RPT, _D), jnp.float32)

    xbar = _dense_call(h, W, q_pad)
    part = _sc_call(xbar.reshape(_NREL * _N, _D), srcs, dsts, zeros_hbm)
    h_next, attn_p = _fuse_call(part.reshape(2 * _NREL, _NPAD, _D),
                                rv_pad, relation_bias)
    rel_attn_global = attn_p[:, 0, :_NREL].sum(axis=0) * (1.0 / _N)
    return (h_next, rel_attn_global)
